# baseline (speedup 1.0000x reference)
import functools

import jax
import jax.numpy as jnp
import numpy as np
from jax.experimental import pallas as pl
from jax.experimental.pallas import tpu as pltpu

N_NODES = 10000
N_EDGES = 320000
DIM1 = 64
B = 64
T = 32
D_MODEL = 512
HEADS = 4
VOCAB = 120


def _ap(x, l):
    y = x @ l["w"]
    if "b" in l:
        y = y + l["b"]
    return y


def _ln(x, p, eps=1e-5):
    m = x.mean(-1, keepdims=True)
    v = x.var(-1, keepdims=True)
    return (x - m) / jnp.sqrt(v + eps) * p["g"] + p["b"]


def _bn(x, g, b, eps=1e-5):
    m = x.mean(0)
    v = x.var(0)
    return (x - m) / jnp.sqrt(v + eps) * g + b


def _gated_gcn(lp, x, s, d, e):
    e_new = _ap(x, lp["A"])[s] + _ap(x, lp["B"])[d] + _ap(e, lp["C"])
    eta = jax.nn.sigmoid(e_new)
    msg = eta * _ap(x, lp["V"])[s]
    num = jax.ops.segment_sum(msg, d, num_segments=N_NODES)
    den = jax.ops.segment_sum(eta, d, num_segments=N_NODES) + 1e-6
    x_new = jax.nn.silu(_ap(x, lp["U"]) + num / den)
    return x_new, e_new


def _egat(lp, x, s, d, e):
    m = jax.nn.leaky_relu(_ap(jnp.concatenate([x[s], x[d], e], axis=-1), lp["W"]), 0.2)
    sc = m @ lp["a"]
    smax = jax.ops.segment_max(sc, d, num_segments=N_NODES)
    smax = jnp.where(jnp.isfinite(smax), smax, 0.0)
    ex = jnp.exp(sc - smax[d])
    den = jax.ops.segment_sum(ex, d, num_segments=N_NODES) + 1e-16
    alpha = ex / den[d]
    return jax.ops.segment_sum(alpha[:, None] * m, d, num_segments=N_NODES)


def _mha(h, lp, pad):
    Bq, Tq, _ = h.shape
    dh = D_MODEL // HEADS
    q = _ap(h, lp["Wq"]).reshape(Bq, Tq, HEADS, dh).transpose(0, 2, 1, 3)
    kk = _ap(h, lp["Wk"]).reshape(Bq, Tq, HEADS, dh).transpose(0, 2, 1, 3)
    v = _ap(h, lp["Wv"]).reshape(Bq, Tq, HEADS, dh).transpose(0, 2, 1, 3)
    logits = jnp.einsum("bhqd,bhkd->bhqk", q, kk) / np.sqrt(dh)
    logits = jnp.where(pad[:, None, None, :], -1e9, logits)
    a = jax.nn.softmax(logits, axis=-1)
    o = jnp.einsum("bhqk,bhkd->bhqd", a, v).transpose(0, 2, 1, 3).reshape(Bq, Tq, D_MODEL)
    return _ap(o, lp["Wo"])


def _enc_layer(h, lp, pad):
    h = _ln(h + _mha(h, lp, pad), lp["ln1"])
    h = _ln(h + _ap(jax.nn.relu(_ap(h, lp["ff1"])), lp["ff2"]), lp["ln2"])
    return h


def _resnet(fea, params):
    for fc, skip in zip(params["res_fcs"], params["res_skips"]):
        fea = jax.nn.leaky_relu(_ap(fea, fc), 0.01) + fea @ skip["w"]
    return _ap(fea, params["res_out"])


def _head_kernel(cat_ref, w_ref, b_ref, o_ref):
    o_ref[...] = cat_ref[...] @ w_ref[...] + b_ref[...]


def _head(cat, w, b):
    return pl.pallas_call(
        _head_kernel,
        out_shape=jax.ShapeDtypeStruct((cat.shape[0], 128), jnp.float32),
    )(cat, w, b)


def kernel(x, edge_attr, glob_feat, frac, human_d, params, edge_index, batch, src):
    s_, d_ = edge_index[0], edge_index[1]
    out_x = jax.nn.softplus(_ln(_ap(jax.nn.silu(_ap(x, params["node_mlp1"])), params["node_mlp2"]), params["node_ln"]))
    out_e = jax.nn.softplus(_ln(_ap(jax.nn.silu(_ap(edge_attr, params["edge_mlp1"])), params["edge_mlp2"]), params["edge_ln"]))
    prev = out_x
    ea = out_e
    for lp in params["ggcn"]:
        out_x, ea = _gated_gcn(lp, out_x, s_, d_, out_e)
        out_x, ea = _gated_gcn(lp, out_x, s_, d_, ea)
    for lp in params["egat"]:
        h = _egat(lp, out_x, s_, d_, ea)
        h = _bn(h, lp["bn_g"], lp["bn_b"])
        out_x = h + prev
        prev = out_x
    gate = jax.nn.sigmoid(_ap(jnp.concatenate([out_x, glob_feat[batch]], axis=-1), params["gate"]))
    out_x = out_x * gate
    pad = (src == 0)
    ht = params["embed"][src] * frac[..., None]
    for lp in params["enc"]:
        ht = _enc_layer(ht, lp, pad)
    out = _resnet(ht, params)
    maskf = (~pad)[..., None].astype(out.dtype)
    out = (out * maskf).sum(axis=1) / maskf.sum(axis=1)
    pooled = jax.ops.segment_sum(out_x, batch, num_segments=B)
    hf = _bn(_ap(human_d, params["human"]), params["human_bn"]["g"], params["human_bn"]["b"])
    cat = jnp.concatenate([pooled, hf, out], axis=1)
    w = params["lin_out"]["w"]
    wp = jnp.zeros((192, 128), jnp.float32).at[:, :1].set(w)
    bp = jnp.zeros((B, 128), jnp.float32).at[:, :1].set(params["lin_out"]["b"][None, :])
    res = _head(cat, wp, bp)[:, :1]
    return res.reshape(-1)


# speedup vs baseline: 3.6603x; 3.6603x over previous
import functools

import jax
import jax.numpy as jnp
import numpy as np
from jax import lax
from jax.experimental import pallas as pl
from jax.experimental.pallas import tpu as pltpu
from jax.experimental.pallas import tpu_sc as plsc

N_NODES = 10000
N_EDGES = 320000
DIM1 = 64
B = 64
T = 32
D_MODEL = 512
HEADS = 4
DH = D_MODEL // HEADS
VOCAB = 120

NW = 32
EPW = N_EDGES // NW
K = 80
NCH = EPW // K
NPS = N_NODES // 16

f32 = jnp.float32


def _full_spec(shape):
    n = len(shape)
    return pl.BlockSpec(shape, lambda *_: (0,) * n)


def _node_mlp_body(x_ref, w1_ref, b1_ref, w2_ref, b2_ref, g_ref, bb_ref, o_ref):
    h = jax.nn.silu(x_ref[...] @ w1_ref[...] + b1_ref[...])
    y = h @ w2_ref[...] + b2_ref[...]
    m = y.mean(-1, keepdims=True)
    v = ((y - m) ** 2).mean(-1, keepdims=True)
    y = (y - m) / jnp.sqrt(v + 1e-5) * g_ref[...] + bb_ref[...]
    o_ref[...] = jnp.maximum(y, 0.0) + jnp.log1p(jnp.exp(-jnp.abs(y)))


def _node_mlp(x, p1, p2, ln):
    n = x.shape[0]
    return pl.pallas_call(
        _node_mlp_body,
        out_shape=jax.ShapeDtypeStruct((n, DIM1), f32),
    )(x, p1["w"], p1["b"][None, :], p2["w"], p2["b"][None, :],
      ln["g"][None, :], ln["b"][None, :])


def _edge_mlp_body(x_ref, w1_ref, b1_ref, w2_ref, b2_ref, g_ref, bb_ref, o_ref):
    _node_mlp_body(x_ref, w1_ref, b1_ref, w2_ref, b2_ref, g_ref, bb_ref, o_ref)


def _edge_mlp(ea, p1, p2, ln):
    blk = 8000
    grid = N_EDGES // blk
    return pl.pallas_call(
        _edge_mlp_body,
        grid=(grid,),
        in_specs=[
            pl.BlockSpec((blk, 16), lambda i: (i, 0)),
            _full_spec((16, DIM1)), _full_spec((1, DIM1)),
            _full_spec((DIM1, DIM1)), _full_spec((1, DIM1)),
            _full_spec((1, DIM1)), _full_spec((1, DIM1)),
        ],
        out_specs=pl.BlockSpec((blk, DIM1), lambda i: (i, 0)),
        out_shape=jax.ShapeDtypeStruct((N_EDGES, DIM1), f32),
    )(ea, p1["w"], p1["b"][None, :], p2["w"], p2["b"][None, :],
      ln["g"][None, :], ln["b"][None, :])


def _mm_body(x_ref, w_ref, b_ref, o_ref):
    o_ref[...] = x_ref[...] @ w_ref[...] + b_ref[...]


def _ggcn_mats_body(x_ref, w_ref, b_ref, p_ref, q_ref, cw_ref, r_ref, rb_ref,
                    t1_ref, t2_ref, ux_ref, rn_ref, rbn_ref):
    y = x_ref[...] @ w_ref[...] + b_ref[...]
    pc = p_ref[...] @ cw_ref[...]
    qc = q_ref[...] @ cw_ref[...]
    t1_ref[...] = jnp.concatenate([y[:, :DIM1] + pc, y[:, DIM1:2 * DIM1]], axis=1)
    t2_ref[...] = jnp.concatenate([y[:, 2 * DIM1:3 * DIM1] + qc,
                                   jnp.zeros_like(qc)], axis=1)
    ux_ref[...] = y[:, 3 * DIM1:]
    rn_ref[...] = r_ref[...] @ cw_ref[...]
    rbn_ref[...] = rb_ref[...] @ cw_ref[...]


def _ggcn_mats(x, wab, bab, p, q, cw, cb, r, rb):
    t1, t2, ux, rn, rbn = pl.pallas_call(
        _ggcn_mats_body,
        out_shape=[
            jax.ShapeDtypeStruct((N_NODES, 128), f32),
            jax.ShapeDtypeStruct((N_NODES, 128), f32),
            jax.ShapeDtypeStruct((N_NODES, DIM1), f32),
            jax.ShapeDtypeStruct((DIM1, DIM1), f32),
            jax.ShapeDtypeStruct((1, DIM1), f32),
        ],
    )(x, wab, bab[None, :], p, q, cw, r, rb)
    return t1, t2, ux, rn, rbn + cb[None, :]


def _egat_mats_body(x_ref, w12_ref, p_ref, q_ref, w3_ref, r_ref, rb_ref, bw_ref,
                    t12_ref, cw_ref, cb_ref):
    y = x_ref[...] @ w12_ref[...]
    pw = p_ref[...] @ w3_ref[...]
    qw = q_ref[...] @ w3_ref[...]
    t12_ref[...] = jnp.concatenate([y[:, :DIM1] + pw, y[:, DIM1:] + qw], axis=1)
    cw_ref[...] = r_ref[...] @ w3_ref[...]
    cb_ref[...] = rb_ref[...] @ w3_ref[...] + bw_ref[...]


def _egat_mats(x, w12, p, q, w3, r, rb, bw):
    return pl.pallas_call(
        _egat_mats_body,
        out_shape=[
            jax.ShapeDtypeStruct((N_NODES, 128), f32),
            jax.ShapeDtypeStruct((DIM1, DIM1), f32),
            jax.ShapeDtypeStruct((1, DIM1), f32),
        ],
    )(x, w12, p, q, w3, r, rb, bw[None, :])


def _node_mm(x, w, b):
    n, k = x.shape
    m = w.shape[1]
    if b is None:
        b = jnp.zeros((m,), f32)
    return pl.pallas_call(
        _mm_body,
        out_shape=jax.ShapeDtypeStruct((n, m), f32),
    )(x, w, b[None, :])


def _edge_mm(e, w, b):
    blk = 16000
    grid = N_EDGES // blk
    return pl.pallas_call(
        _mm_body,
        grid=(grid,),
        in_specs=[
            pl.BlockSpec((blk, DIM1), lambda i: (i, 0)),
            _full_spec((DIM1, DIM1)), _full_spec((1, DIM1)),
        ],
        out_specs=pl.BlockSpec((blk, DIM1), lambda i: (i, 0)),
        out_shape=jax.ShapeDtypeStruct((N_EDGES, DIM1), f32),
    )(e, w, b[None, :])


def _node_upd_body(p_ref, ux_ref, o_ref):
    num = p_ref[0, :, :DIM1] + p_ref[1, :, :DIM1]
    den = p_ref[0, :, DIM1:] + p_ref[1, :, DIM1:] + 1e-6
    z = ux_ref[...] + num / den
    o_ref[...] = z * (1.0 / (1.0 + jnp.exp(-z)))


def _node_upd(part, ux):
    return pl.pallas_call(
        _node_upd_body,
        out_shape=jax.ShapeDtypeStruct((N_NODES, DIM1), f32),
    )(part, ux)


def _egat_node_body(p_ref, prev_ref, g_ref, b_ref, o_ref):
    num = p_ref[0, :, :DIM1] + p_ref[1, :, :DIM1]
    den = p_ref[0, :, DIM1:DIM1 + 1] + p_ref[1, :, DIM1:DIM1 + 1] + 1e-16
    h = num / den
    m = h.mean(0, keepdims=True)
    v = ((h - m) ** 2).mean(0, keepdims=True)
    h = (h - m) / jnp.sqrt(v + 1e-5) * g_ref[...] + b_ref[...]
    o_ref[...] = h + prev_ref[...]


def _egat_node(part, prev, g, b):
    return pl.pallas_call(
        _egat_node_body,
        out_shape=jax.ShapeDtypeStruct((N_NODES, DIM1), f32),
    )(part, prev, g[None, :], b[None, :])


def _embed_body(src_ref, frac_ref, emb_ref, o_ref):
    oh = (src_ref[...] == lax.broadcasted_iota(jnp.int32, (B * T, VOCAB), 1)
          ).astype(f32)
    o_ref[...] = (oh @ emb_ref[...]) * frac_ref[...]


def _embed(src2, frac2, emb):
    return pl.pallas_call(
        _embed_body,
        out_shape=jax.ShapeDtypeStruct((B * T, D_MODEL), f32),
    )(src2, frac2, emb)


def _qkv_body(h_ref, w_ref, b_ref, o_ref):
    o_ref[...] = h_ref[...] @ w_ref[...] + b_ref[...]


def _qkv(h, w, b):
    blk = 512
    return pl.pallas_call(
        _qkv_body,
        grid=(B * T // blk,),
        in_specs=[
            pl.BlockSpec((blk, D_MODEL), lambda i: (i, 0)),
            _full_spec((D_MODEL, 3 * D_MODEL)), _full_spec((1, 3 * D_MODEL)),
        ],
        out_specs=pl.BlockSpec((blk, 3 * D_MODEL), lambda i: (i, 0)),
        out_shape=jax.ShapeDtypeStruct((B * T, 3 * D_MODEL), f32),
    )(h, w, b[None, :])


def _attn_body(qkv_ref, src_ref, o_ref):
    pad = (src_ref[0, :, :] == 0)
    scale = 1.0 / np.sqrt(DH)
    for hd in range(HEADS):
        q = qkv_ref[0][:, hd * DH:(hd + 1) * DH]
        k = qkv_ref[0][:, D_MODEL + hd * DH:D_MODEL + (hd + 1) * DH]
        v = qkv_ref[0][:, 2 * D_MODEL + hd * DH:2 * D_MODEL + (hd + 1) * DH]
        logits = lax.dot_general(q, k, (((1,), (1,)), ((), ()))) * scale
        logits = jnp.where(pad, -1e9, logits)
        mx = logits.max(-1, keepdims=True)
        p = jnp.exp(logits - mx)
        p = p / p.sum(-1, keepdims=True)
        o_ref[0, :, hd * DH:(hd + 1) * DH] = p @ v


def _attn(qkv3, src3):
    return pl.pallas_call(
        _attn_body,
        grid=(B,),
        in_specs=[
            pl.BlockSpec((1, T, 3 * D_MODEL), lambda i: (i, 0, 0)),
            pl.BlockSpec((1, 1, T), lambda i: (i, 0, 0)),
        ],
        out_specs=pl.BlockSpec((1, T, D_MODEL), lambda i: (i, 0, 0)),
        out_shape=jax.ShapeDtypeStruct((B, T, D_MODEL), f32),
    )(qkv3, src3)


def _ln_in(y, g, b):
    m = y.mean(-1, keepdims=True)
    v = ((y - m) ** 2).mean(-1, keepdims=True)
    return (y - m) / jnp.sqrt(v + 1e-5) * g + b


def _postattn_body(h_ref, o_ref, wo_ref, bo_ref, g1_ref, b1_ref,
                   w1_ref, bb1_ref, w2_ref, bb2_ref, g2_ref, b2_ref, out_ref):
    h1 = _ln_in(h_ref[...] + o_ref[...] @ wo_ref[...] + bo_ref[...],
                g1_ref[...], b1_ref[...])
    ff = jnp.maximum(h1 @ w1_ref[...] + bb1_ref[...], 0.0) @ w2_ref[...] + bb2_ref[...]
    out_ref[...] = _ln_in(h1 + ff, g2_ref[...], b2_ref[...])


def _postattn(h, o, lp):
    blk = 512
    D4 = 4 * D_MODEL
    return pl.pallas_call(
        _postattn_body,
        grid=(B * T // blk,),
        in_specs=[
            pl.BlockSpec((blk, D_MODEL), lambda i: (i, 0)),
            pl.BlockSpec((blk, D_MODEL), lambda i: (i, 0)),
            _full_spec((D_MODEL, D_MODEL)), _full_spec((1, D_MODEL)),
            _full_spec((1, D_MODEL)), _full_spec((1, D_MODEL)),
            _full_spec((D_MODEL, D4)), _full_spec((1, D4)),
            _full_spec((D4, D_MODEL)), _full_spec((1, D_MODEL)),
            _full_spec((1, D_MODEL)), _full_spec((1, D_MODEL)),
        ],
        out_specs=pl.BlockSpec((blk, D_MODEL), lambda i: (i, 0)),
        out_shape=jax.ShapeDtypeStruct((B * T, D_MODEL), f32),
    )(h, o, lp["Wo"]["w"], lp["Wo"]["b"][None, :],
      lp["ln1"]["g"][None, :], lp["ln1"]["b"][None, :],
      lp["ff1"]["w"], lp["ff1"]["b"][None, :],
      lp["ff2"]["w"], lp["ff2"]["b"][None, :],
      lp["ln2"]["g"][None, :], lp["ln2"]["b"][None, :])


def _resnet_body(x_ref, maskb_ref, *refs):
    out_ref = refs[-1]
    nlay = (len(refs) - 3) // 3
    fea = x_ref[...]
    for i in range(nlay):
        w, bb, sk = refs[3 * i], refs[3 * i + 1], refs[3 * i + 2]
        y = fea @ w[...] + bb[...]
        fea = jnp.where(y > 0, y, 0.01 * y) + fea @ sk[...]
    out_ref[...] = (fea @ refs[-3][...] + refs[-2][...]) * maskb_ref[...]


def _resnet(x, maskb, fcs, skips, ro):
    blk = 512
    args = [x, maskb]
    specs = [pl.BlockSpec((blk, D_MODEL), lambda i: (i, 0)),
             pl.BlockSpec((blk, DIM1), lambda i: (i, 0))]
    for fc, sk in zip(fcs, skips):
        args += [fc["w"], fc["b"][None, :], sk["w"]]
        specs += [_full_spec(fc["w"].shape), _full_spec((1, fc["w"].shape[1])),
                  _full_spec(sk["w"].shape)]
    args += [ro["w"], ro["b"][None, :]]
    specs += [_full_spec(ro["w"].shape), _full_spec((1, DIM1))]
    return pl.pallas_call(
        _resnet_body,
        grid=(B * T // blk,),
        in_specs=specs,
        out_specs=pl.BlockSpec((blk, DIM1), lambda i: (i, 0)),
        out_shape=jax.ShapeDtypeStruct((B * T, DIM1), f32),
    )(*args)


def _gatepool_body(x_ref, batch_ref, gg_ref, wg_ref, bg_ref, o_ref):
    oh = (batch_ref[...] == lax.broadcasted_iota(jnp.int32, (N_NODES, B), 1)
          ).astype(f32)
    gate = x_ref[...] @ wg_ref[...] + oh @ gg_ref[...] + bg_ref[...]
    gate = 1.0 / (1.0 + jnp.exp(-gate))
    gx = x_ref[...] * gate
    o_ref[...] = lax.dot_general(oh, gx, (((0,), (0,)), ((), ())))


def _gatepool(out_x, batch2, gg, wg1, bg):
    return pl.pallas_call(
        _gatepool_body,
        out_shape=jax.ShapeDtypeStruct((B, DIM1), f32),
    )(out_x, batch2, gg, wg1, bg[None, :])


def _final_body(res_ref, mask_ref, pooled_ref, hd_ref, wh_ref, bh_ref,
                g_ref, b_ref, w0_ref, w1_ref, w2_ref, bo_ref, o_ref):
    osum = res_ref[...].sum(axis=1)
    cnt = mask_ref[...].sum(axis=1)
    ot = osum / cnt
    hf = hd_ref[...] @ wh_ref[...] + bh_ref[...]
    m = hf.mean(0, keepdims=True)
    v = ((hf - m) ** 2).mean(0, keepdims=True)
    hf = (hf - m) / jnp.sqrt(v + 1e-5) * g_ref[...] + b_ref[...]
    o_ref[...] = (pooled_ref[...] @ w0_ref[...] + hf @ w1_ref[...]
                  + ot @ w2_ref[...] + bo_ref[...])


def _final(res3, mask3, pooled, human_d, ph, pbn, w, bo):
    return pl.pallas_call(
        _final_body,
        out_shape=jax.ShapeDtypeStruct((B, 1), f32),
    )(res3, mask3, pooled, human_d, ph["w"], ph["b"][None, :],
      pbn["g"][None, :], pbn["b"][None, :],
      w[:DIM1], w[DIM1:2 * DIM1], w[2 * DIM1:], bo[None, :])


_SC_MESH = None


def _sc_mesh():
    global _SC_MESH
    if _SC_MESH is None:
        _SC_MESH = plsc.VectorSubcoreMesh(core_axis_name="c", subcore_axis_name="s")
    return _SC_MESH


_DR = 40
_NQCH = N_NODES // _DR


def _gtake(v, idx):
    return v.at[idx].get(mode="promise_in_bounds")


def _allsum(v):
    lane = lax.iota(jnp.int32, 16)
    for k in (1, 2, 4, 8):
        v = v + _gtake(v, lane ^ k)
    return v


def _allmax(v):
    lane = lax.iota(jnp.int32, 16)
    for k in (1, 2, 4, 8):
        v = jnp.maximum(v, _gtake(v, lane ^ k))
    return v


def _bcast_lane(v, i):
    return _gtake(v, jnp.full((16,), i, jnp.int32))


def _nq_of(sid):
    lo = _NQCH // 16
    rem = _NQCH % 16
    return jnp.where(sid < rem, lo + 1, lo)


def _zero_accum(buf, accum, sid, width):
    def zrow(i, _):
        for f in range(width // 16):
            buf[i, pl.ds(f * 16, 16)] = jnp.zeros((16,), f32)
        return _

    lax.fori_loop(0, K, zrow, None)
    sub = buf.at[pl.ds(0, _DR)]

    def zchunk(q, _):
        c = q * 16 + sid
        pltpu.sync_copy(sub, accum.at[pl.ds(c * _DR, _DR)])
        return _

    lax.fori_loop(0, _nq_of(sid), zchunk, None)


def _drain_accum(buf, accum, part, cid, sid):
    sub = buf.at[pl.ds(0, _DR)]

    def dchunk(q, _):
        c = q * 16 + sid
        rows = pl.ds(c * _DR, _DR)
        pltpu.sync_copy(accum.at[rows], sub)
        pltpu.sync_copy(sub, part.at[cid, rows])
        return _

    lax.fori_loop(0, _nq_of(sid), dchunk, None)


def _unpack_sd(pbuf, sidx, didx):
    for q in range(K // 16):
        qs = pl.ds(q * 16, 16)
        v = pbuf[qs]
        sidx[qs] = jnp.right_shift(v, 14)
        didx[qs] = v & 16383


def _ggcn_sc_body(t1, t2, ce, sd3, part,
                  pbuf, sidx, didx, g1, g2, ceb, scat, accum,
                  sem1, sem2, sem3):
    cid = lax.axis_index("c")
    sid = lax.axis_index("s")
    wid = sid * 2 + cid
    _zero_accum(scat, accum, sid, 128)
    plsc.subcore_barrier()
    base = wid * EPW

    def chunk(j, _):
        eb = base + j * K
        pltpu.sync_copy(sd3.at[wid, j], pbuf)
        _unpack_sd(pbuf, sidx, didx)
        c1 = pltpu.async_copy(t1.at[sidx], g1, sem1)
        c2 = pltpu.async_copy(t2.at[didx], g2, sem2)
        c3 = pltpu.async_copy(ce.at[pl.ds(eb, K)], ceb, sem3)
        c1.wait()
        c2.wait()
        c3.wait()

        def row(r, _):
            for f in range(4):
                fs = pl.ds(f * 16, 16)
                en = g1[r, fs] + g2[r, fs] + ceb[r, fs]
                eta = 1.0 / (1.0 + jnp.exp(-en))
                scat[r, fs] = eta * g1[r, pl.ds(64 + f * 16, 16)]
                scat[r, pl.ds(64 + f * 16, 16)] = eta
            return _

        lax.fori_loop(0, K, row, None, unroll=2)
        pltpu.sync_copy(scat, accum.at[didx], add=True)
        return _

    lax.fori_loop(0, NCH, chunk, None)
    plsc.subcore_barrier()
    _drain_accum(scat, accum, part, cid, sid)


def _ggcn_sc(t1, t2, ce, sd3):
    fn = pl.kernel(
        _ggcn_sc_body,
        out_type=[
            jax.ShapeDtypeStruct((2, N_NODES, 128), f32),
        ],
        mesh=_sc_mesh(),
        scratch_types=[
            pltpu.VMEM((K,), jnp.int32),
            pltpu.VMEM((K,), jnp.int32),
            pltpu.VMEM((K,), jnp.int32),
            pltpu.VMEM((K, 128), f32),
            pltpu.VMEM((K, 128), f32),
            pltpu.VMEM((K, DIM1), f32),
            pltpu.VMEM((K, 128), f32),
            pltpu.VMEM_SHARED((N_NODES, 128), f32),
            pltpu.SemaphoreType.DMA,
            pltpu.SemaphoreType.DMA,
            pltpu.SemaphoreType.DMA,
        ],
    )
    return fn(t1, t2, ce, sd3)[0]


def _egat_p1_body(t1, t2, ew3, sd3, avec, m_out, sc_out, pmax,
                  pbuf, sidx, didx, g1, g2, ceb, mb, scb, ab, rb,
                  sem1, sem2, sem3):
    cid = lax.axis_index("c")
    sid = lax.axis_index("s")
    wid = sid * 2 + cid
    pltpu.sync_copy(avec, ab)
    base = wid * EPW
    lane = lax.iota(jnp.int32, 16)

    def chunk(j, rmax):
        eb = base + j * K
        pltpu.sync_copy(sd3.at[wid, j], pbuf)
        _unpack_sd(pbuf, sidx, didx)
        c1 = pltpu.async_copy(t1.at[sidx], g1, sem1)
        c2 = pltpu.async_copy(t2.at[didx], g2, sem2)
        c3 = pltpu.async_copy(ew3.at[pl.ds(eb, K)], ceb, sem3)
        c1.wait()
        c2.wait()
        c3.wait()

        def grp(g, rmax):
            acc = jnp.zeros((16,), f32)
            for rr in range(16):
                r = g * 16 + rr
                v = jnp.zeros((16,), f32)
                for f in range(4):
                    fs = pl.ds(f * 16, 16)
                    pre = g1[r, fs] + g2[r, pl.ds(64 + f * 16, 16)] + ceb[r, fs]
                    mm = jnp.maximum(pre, 0.2 * pre)
                    mb[r, fs] = mm
                    v = v + mm * ab[fs]
                acc = jnp.where(lane == rr, _allsum(v), acc)
            scb[pl.ds(g * 16, 16)] = acc
            return jnp.maximum(rmax, acc)

        rmax = lax.fori_loop(0, K // 16, grp, rmax)
        pltpu.sync_copy(mb, m_out.at[pl.ds(eb, K)])
        pltpu.sync_copy(scb, sc_out.at[pl.ds(eb, K)])
        return rmax

    rmax = lax.fori_loop(0, NCH, chunk, jnp.full((16,), -1e30, f32))
    rb[...] = rmax
    pltpu.sync_copy(rb, pmax.at[wid])


def _egat_p1(t1, t2, ew3, sd3, avec):
    fn = pl.kernel(
        _egat_p1_body,
        out_type=[
            jax.ShapeDtypeStruct((N_EDGES, DIM1), f32),
            jax.ShapeDtypeStruct((N_EDGES,), f32),
            jax.ShapeDtypeStruct((NW, 16), f32),
        ],
        mesh=_sc_mesh(),
        scratch_types=[
            pltpu.VMEM((K,), jnp.int32),
            pltpu.VMEM((K,), jnp.int32),
            pltpu.VMEM((K,), jnp.int32),
            pltpu.VMEM((K, 128), f32),
            pltpu.VMEM((K, 128), f32),
            pltpu.VMEM((K, DIM1), f32),
            pltpu.VMEM((K, DIM1), f32),
            pltpu.VMEM((K,), f32),
            pltpu.VMEM((DIM1,), f32),
            pltpu.VMEM((16,), f32),
            pltpu.SemaphoreType.DMA,
            pltpu.SemaphoreType.DMA,
            pltpu.SemaphoreType.DMA,
        ],
    )
    return fn(t1, t2, ew3, sd3, avec)


def _egat_p2_body(m_in, sc_in, pmax, sd3, part,
                  pbuf, didx, mb, scb, scat, pmb, accum,
                  sem1, sem2, sem3):
    cid = lax.axis_index("c")
    sid = lax.axis_index("s")
    wid = sid * 2 + cid
    _zero_accum(scat, accum, sid, 128)
    plsc.subcore_barrier()
    pltpu.sync_copy(pmax, pmb)
    gm = jnp.full((16,), -1e30, f32)
    for i in range(NW):
        gm = jnp.maximum(gm, pmb[i])
    gmax = _allmax(gm)
    base = wid * EPW
    lane = lax.iota(jnp.int32, 16)

    def chunk(j, _):
        eb = base + j * K
        pltpu.sync_copy(sd3.at[wid, j], pbuf)
        for q in range(K // 16):
            qs = pl.ds(q * 16, 16)
            didx[qs] = pbuf[qs] & 16383
        c1 = pltpu.async_copy(m_in.at[pl.ds(eb, K)], mb, sem1)
        c2 = pltpu.async_copy(sc_in.at[pl.ds(eb, K)], scb, sem2)
        c1.wait()
        c2.wait()

        def grp(q, _):
            exv = jnp.exp(scb[pl.ds(q * 16, 16)] - gmax)
            for rr in range(16):
                r = q * 16 + rr
                exs = _bcast_lane(exv, rr)
                for f in range(4):
                    fs = pl.ds(f * 16, 16)
                    scat[r, fs] = mb[r, fs] * exs
                scat[r, pl.ds(64, 16)] = jnp.where(lane == 0, exs, 0.0)
            return _

        lax.fori_loop(0, K // 16, grp, None)
        pltpu.sync_copy(scat, accum.at[didx], add=True)
        return _

    lax.fori_loop(0, NCH, chunk, None)
    plsc.subcore_barrier()
    _drain_accum(scat, accum, part, cid, sid)


def _egat_p2(m_in, sc_in, pmax, sd3):
    fn = pl.kernel(
        _egat_p2_body,
        out_type=[
            jax.ShapeDtypeStruct((2, N_NODES, 128), f32),
        ],
        mesh=_sc_mesh(),
        scratch_types=[
            pltpu.VMEM((K,), jnp.int32),
            pltpu.VMEM((K,), jnp.int32),
            pltpu.VMEM((K, DIM1), f32),
            pltpu.VMEM((K,), f32),
            pltpu.VMEM((K, 128), f32),
            pltpu.VMEM((NW, 16), f32),
            pltpu.VMEM_SHARED((N_NODES, 128), f32),
            pltpu.SemaphoreType.DMA,
            pltpu.SemaphoreType.DMA,
            pltpu.SemaphoreType.DMA,
        ],
    )
    return fn(m_in, sc_in, pmax, sd3)[0]


def kernel(x, edge_attr, glob_feat, frac, human_d, params, edge_index, batch, src):
    s_ = edge_index[0].astype(jnp.int32)
    d_ = edge_index[1].astype(jnp.int32)
    sd3 = (s_ * 16384 + d_).reshape(NW, NCH, K)

    out_x = _node_mlp(x, params["node_mlp1"], params["node_mlp2"], params["node_ln"])
    out_e = _edge_mlp(edge_attr, params["edge_mlp1"], params["edge_mlp2"], params["edge_ln"])

    prev = out_x
    zn = jnp.zeros((N_NODES, DIM1), f32)
    eye = jnp.eye(DIM1, dtype=f32)
    zr = jnp.zeros((1, DIM1), f32)
    for lp in params["ggcn"]:
        wab = jnp.concatenate(
            [lp["A"]["w"], lp["V"]["w"], lp["B"]["w"], lp["U"]["w"]], axis=1)
        bab = jnp.concatenate(
            [lp["A"]["b"], lp["V"]["b"], lp["B"]["b"], lp["U"]["b"]])
        P, Q, R, rb = zn, zn, eye, zr
        for _ in range(2):
            t1, t2, ux, R, rb = _ggcn_mats(
                out_x, wab, bab, P, Q, lp["C"]["w"], lp["C"]["b"], R, rb)
            ce = _edge_mm(out_e, R, rb[0])
            part = _ggcn_sc(t1, t2, ce, sd3)
            out_x = _node_upd(part, ux)
            P = t1[:, :DIM1]
            Q = t2[:, :DIM1]

    for lp in params["egat"]:
        ww = lp["W"]["w"]
        w12 = jnp.concatenate([ww[:DIM1], ww[DIM1:2 * DIM1]], axis=1)
        t12, cw3, cb3 = _egat_mats(out_x, w12, P, Q, ww[2 * DIM1:], R, rb,
                                   lp["W"]["b"])
        ew3 = _edge_mm(out_e, cw3, cb3[0])
        m_e, sc_e, pmax = _egat_p1(t12, t12, ew3, sd3, lp["a"])
        part = _egat_p2(m_e, sc_e, pmax, sd3)
        out_x = _egat_node(part, prev, lp["bn_g"], lp["bn_b"])
        prev = out_x

    gg = glob_feat @ params["gate"]["w"][DIM1:]
    pooled = _gatepool(out_x, batch.astype(jnp.int32).reshape(N_NODES, 1),
                       gg, params["gate"]["w"][:DIM1], params["gate"]["b"])

    src2 = src.astype(jnp.int32).reshape(B * T, 1)
    frac2 = frac.reshape(B * T, 1)
    ht = _embed(src2, frac2, params["embed"])
    src3 = src.astype(jnp.int32).reshape(B, 1, T)
    for lp in params["enc"]:
        wqkv = jnp.concatenate([lp["Wq"]["w"], lp["Wk"]["w"], lp["Wv"]["w"]], axis=1)
        bqkv = jnp.concatenate([lp["Wq"]["b"], lp["Wk"]["b"], lp["Wv"]["b"]])
        qkv = _qkv(ht, wqkv, bqkv)
        o = _attn(qkv.reshape(B, T, 3 * D_MODEL), src3)
        ht = _postattn(ht, o.reshape(B * T, D_MODEL), lp)

    maskb = jnp.broadcast_to(
        (src.reshape(B * T, 1) != 0).astype(f32), (B * T, DIM1))
    res = _resnet(ht, maskb, params["res_fcs"], params["res_skips"], params["res_out"])
    res3 = res.reshape(B, T, DIM1)
    mask3 = maskb.reshape(B, T, DIM1)

    out = _final(res3, mask3, pooled, human_d, params["human"],
                 params["human_bn"], params["lin_out"]["w"], params["lin_out"]["b"])
    return out.reshape(-1)
